# baseline (speedup 1.0000x reference)
import functools

import jax
import jax.numpy as jnp
from jax import lax
from jax.experimental import pallas as pl
from jax.experimental.pallas import tpu as pltpu
from jax.experimental.pallas import tpu_sc as plsc

_REF_PREC = jax.lax.Precision.DEFAULT
_EXACT_PREC = jax.lax.Precision.HIGHEST

_TP = 640
_KP = 456


def _fused_body(x_ref, w_ref, b_ref, m_ref, wp_ref, bp_ref,
                y_ref, uidx_ref, *, T, C, H, HD, K, NH, SCALE):
    xb = x_ref[0]
    qkv = jnp.dot(xb, w_ref[...], preferred_element_type=jnp.float32,
                  precision=_REF_PREC)
    qkv = qkv + b_ref[...]

    kpart = qkv[:, C:2 * C]
    hs = jnp.dot(kpart * kpart, m_ref[...],
                 preferred_element_type=jnp.float32,
                 precision=_EXACT_PREC)
    hmag = jnp.sqrt(hs)
    nrm = jnp.sqrt(jnp.sum(hmag * hmag, axis=1, keepdims=True))
    phi = hmag / jnp.clip(nrm, 1e-12, None)

    meanp = jnp.mean(phi, axis=0, keepdims=True)
    score = jnp.sum(phi * meanp, axis=1, keepdims=True)
    row_id = jax.lax.broadcasted_iota(jnp.int32, (T, 1), 0)
    score = jnp.where(row_id == 0, -jnp.inf, score)
    score_r = jnp.transpose(score)

    ii = jax.lax.broadcasted_iota(jnp.int32, (T, T), 0)
    jj = jax.lax.broadcasted_iota(jnp.int32, (T, T), 1)
    ltr = (score < score_r).astype(jnp.float32)
    eqr = ((score == score_r) & (ii < jj)).astype(jnp.float32)
    rank_row = jnp.sum(ltr + eqr, axis=0, keepdims=True)
    keep_row = rank_row < K

    upper = (ii <= jj).astype(jnp.float32)
    pos_row = jnp.dot(keep_row.astype(jnp.float32), upper,
                      preferred_element_type=jnp.float32) - 1.0

    sim = jax.lax.dot_general(phi, phi, (((1,), (1,)), ((), ())),
                              preferred_element_type=jnp.float32,
                              precision=_REF_PREC)
    simm = jnp.where(keep_row, sim, -jnp.inf)
    mx = jnp.max(simm, axis=1, keepdims=True)
    cand = jnp.where(simm == mx, pos_row, jnp.float32(1e9))
    assign_f = jnp.min(cand, axis=1, keepdims=True)
    assign_row = jnp.transpose(assign_f).astype(jnp.int32)

    b = pl.program_id(0)
    pad_z = jnp.zeros((1, _TP - T), jnp.int32)
    uidx_ref[0] = jnp.concatenate([assign_row + b * _KP, pad_z], axis=1)

    kk = jax.lax.broadcasted_iota(jnp.int32, (K, T), 0)
    sel = (kk == assign_row).astype(jnp.float32)
    mq = jnp.dot(sel, qkv, preferred_element_type=jnp.float32,
                 precision=_EXACT_PREC)
    den = jnp.sum(sel, axis=1, keepdims=True)
    dclip = jnp.clip(den, 1e-12, None)

    ys = []
    for h in range(H):
        qh = mq[:, h * HD:(h + 1) * HD] / dclip * SCALE
        kh = mq[:, C + h * HD:C + (h + 1) * HD] / dclip
        vh = mq[:, 2 * C + h * HD:2 * C + (h + 1) * HD] / dclip
        lg = jax.lax.dot_general(qh, kh, (((1,), (1,)), ((), ())),
                                 preferred_element_type=jnp.float32,
                                 precision=_REF_PREC)
        mxl = jnp.max(lg, axis=1, keepdims=True)
        e = jnp.exp(lg - mxl)
        p = e / jnp.sum(e, axis=1, keepdims=True)
        ys.append(jnp.dot(p, vh, preferred_element_type=jnp.float32,
                          precision=_REF_PREC))
    y = jnp.concatenate(ys, axis=1)
    y = jnp.dot(y, wp_ref[...], preferred_element_type=jnp.float32,
                precision=_REF_PREC)
    y = y + bp_ref[...]
    y_ref[0] = jnp.concatenate(
        [y, jnp.zeros((_KP - K, C), jnp.float32)], axis=0)


def _fused_call(x, W_qkv, b_qkv, W_proj, b_proj, H, R):
    B, T, C = x.shape
    HD = C // H
    K = T - R
    NH = 128
    SCALE = 1.0 / (HD ** 0.5)
    f32 = jnp.float32

    head_onehot = (jnp.arange(C)[:, None] // HD ==
                   jnp.arange(NH)[None, :]).astype(f32)
    b_qkv2 = b_qkv.reshape(1, 3 * C)
    b_proj2 = b_proj.reshape(1, C)

    return pl.pallas_call(
        functools.partial(_fused_body, T=T, C=C, H=H, HD=HD, K=K, NH=NH,
                          SCALE=SCALE),
        grid=(B,),
        in_specs=[
            pl.BlockSpec((1, T, C), lambda b: (b, 0, 0)),
            pl.BlockSpec((C, 3 * C), lambda b: (0, 0)),
            pl.BlockSpec((1, 3 * C), lambda b: (0, 0)),
            pl.BlockSpec((C, NH), lambda b: (0, 0)),
            pl.BlockSpec((C, C), lambda b: (0, 0)),
            pl.BlockSpec((1, C), lambda b: (0, 0)),
        ],
        out_specs=[
            pl.BlockSpec((1, _KP, C), lambda b: (b, 0, 0)),
            pl.BlockSpec((1, 1, _TP), lambda b: (b, 0, 0)),
        ],
        out_shape=[
            jax.ShapeDtypeStruct((B, _KP, C), f32),
            jax.ShapeDtypeStruct((B, 1, _TP), jnp.int32),
        ],
        compiler_params=pltpu.CompilerParams(
            dimension_semantics=("arbitrary",)),
    )(x, W_qkv, b_qkv2, head_onehot, W_proj, b_proj2)


def _sc_unmerge_call(ytab, uidx_flat, B):
    C = ytab.shape[1]
    NC = 2
    CH = 80
    n_ch = _TP // CH
    mesh = plsc.VectorSubcoreMesh(core_axis_name="c", subcore_axis_name="s")

    @functools.partial(
        pl.kernel, mesh=mesh,
        out_type=jax.ShapeDtypeStruct((B, _TP, C), jnp.float32),
        scratch_types=[
            pltpu.VMEM((_TP,), jnp.int32),
            pltpu.VMEM((CH, C), jnp.float32),
            pltpu.SemaphoreType.DMA,
        ],
    )
    def unmerge_k(ytab_hbm, uidx_hbm, out_hbm, idxb_v, rows_v, sem):
        cid = lax.axis_index("c")
        sid = lax.axis_index("s")
        b = sid * NC + cid
        pltpu.sync_copy(
            uidx_hbm.at[pl.ds(pl.multiple_of(b * _TP, 8), _TP)], idxb_v)
        for c in range(n_ch):
            pltpu.async_copy(ytab_hbm.at[idxb_v.at[pl.ds(c * CH, CH)]],
                             rows_v, sem).wait()
            pltpu.sync_copy(rows_v, out_hbm.at[b, pl.ds(c * CH, CH)])

    return unmerge_k(ytab, uidx_flat)


def _run(x, W_qkv, b_qkv, W_proj, b_proj, H, R):
    B, T, C = x.shape
    y, uidx = _fused_call(x, W_qkv, b_qkv, W_proj, b_proj, H, R)
    out_full = _sc_unmerge_call(y.reshape(B * _KP, C), uidx.reshape(-1), B)
    return out_full[:, :T, :]


def kernel(x, W_qkv, b_qkv, W_proj, b_proj, layer_idx, total_layers):
    return _run(x, W_qkv, b_qkv, W_proj, b_proj, H=12, R=128)


# speedup vs baseline: 1.1279x; 1.1279x over previous
import functools

import jax
import jax.numpy as jnp
from jax import lax
from jax.experimental import pallas as pl
from jax.experimental.pallas import tpu as pltpu
from jax.experimental.pallas import tpu_sc as plsc

_REF_PREC = jax.lax.Precision.DEFAULT
_EXACT_PREC = jax.lax.Precision.HIGHEST

_TP = 640
_KP = 456


def _fused_body(x_ref, w_ref, b_ref, m_ref, wp_ref, bp_ref,
                y_ref, uidx_ref, *, T, C, H, HD, K, NH, SCALE):
    xb = x_ref[0]
    qkv = jnp.dot(xb, w_ref[...], preferred_element_type=jnp.float32,
                  precision=_REF_PREC)
    qkv = qkv + b_ref[...]

    kpart = qkv[:, C:2 * C]
    hs = jnp.dot(kpart * kpart, m_ref[...],
                 preferred_element_type=jnp.float32,
                 precision=_EXACT_PREC)
    hmag = jnp.sqrt(hs)
    nrm = jnp.sqrt(jnp.sum(hmag * hmag, axis=1, keepdims=True))
    phi = hmag / jnp.clip(nrm, 1e-12, None)

    meanp = jnp.mean(phi, axis=0, keepdims=True)
    score = jnp.sum(phi * meanp, axis=1, keepdims=True)
    row_id = jax.lax.broadcasted_iota(jnp.int32, (T, 1), 0)
    score = jnp.where(row_id == 0, -jnp.inf, score)
    score_r = jnp.transpose(score)

    ii = jax.lax.broadcasted_iota(jnp.int32, (T, T), 0)
    jj = jax.lax.broadcasted_iota(jnp.int32, (T, T), 1)
    ltr = (score < score_r).astype(jnp.float32)
    eqr = ((score == score_r) & (ii < jj)).astype(jnp.float32)
    rank_row = jnp.sum(ltr + eqr, axis=0, keepdims=True)
    keep_row = rank_row < K

    upper = (ii <= jj).astype(jnp.float32)
    pos_row = jnp.dot(keep_row.astype(jnp.float32), upper,
                      preferred_element_type=jnp.float32) - 1.0

    sim = jax.lax.dot_general(phi, phi, (((1,), (1,)), ((), ())),
                              preferred_element_type=jnp.float32,
                              precision=_REF_PREC)
    simm = jnp.where(keep_row, sim, -jnp.inf)
    mx = jnp.max(simm, axis=1, keepdims=True)
    cand = jnp.where(simm == mx, pos_row, jnp.float32(1e9))
    assign_f = jnp.min(cand, axis=1, keepdims=True)
    assign_row = jnp.transpose(assign_f).astype(jnp.int32)

    b = pl.program_id(0)
    pad_z = jnp.zeros((1, _TP - T), jnp.int32)
    uidx_ref[0] = jnp.concatenate([assign_row + b * _KP, pad_z], axis=1)

    kk = jax.lax.broadcasted_iota(jnp.int32, (K, T), 0)
    sel = (kk == assign_row).astype(jnp.float32)
    mq = jnp.dot(sel, qkv, preferred_element_type=jnp.float32,
                 precision=_EXACT_PREC)
    den = jnp.sum(sel, axis=1, keepdims=True)
    dclip = jnp.clip(den, 1e-12, None)

    ys = []
    for h in range(H):
        qh = mq[:, h * HD:(h + 1) * HD] / dclip * SCALE
        kh = mq[:, C + h * HD:C + (h + 1) * HD] / dclip
        vh = mq[:, 2 * C + h * HD:2 * C + (h + 1) * HD] / dclip
        lg = jax.lax.dot_general(qh, kh, (((1,), (1,)), ((), ())),
                                 preferred_element_type=jnp.float32,
                                 precision=_REF_PREC)
        mxl = jnp.max(lg, axis=1, keepdims=True)
        e = jnp.exp(lg - mxl)
        p = e / jnp.sum(e, axis=1, keepdims=True)
        ys.append(jnp.dot(p, vh, preferred_element_type=jnp.float32,
                          precision=_REF_PREC))
    y = jnp.concatenate(ys, axis=1)
    y = jnp.dot(y, wp_ref[...], preferred_element_type=jnp.float32,
                precision=_REF_PREC)
    y = y + bp_ref[...]
    y_ref[0] = jnp.concatenate(
        [y, jnp.zeros((_KP - K, C), jnp.float32)], axis=0)


def _fused_call(x, W_qkv, b_qkv, W_proj, b_proj, H, R):
    B, T, C = x.shape
    HD = C // H
    K = T - R
    NH = 128
    SCALE = 1.0 / (HD ** 0.5)
    f32 = jnp.float32

    head_onehot = (jnp.arange(C)[:, None] // HD ==
                   jnp.arange(NH)[None, :]).astype(f32)
    b_qkv2 = b_qkv.reshape(1, 3 * C)
    b_proj2 = b_proj.reshape(1, C)

    return pl.pallas_call(
        functools.partial(_fused_body, T=T, C=C, H=H, HD=HD, K=K, NH=NH,
                          SCALE=SCALE),
        grid=(B,),
        in_specs=[
            pl.BlockSpec((1, T, C), lambda b: (b, 0, 0)),
            pl.BlockSpec((C, 3 * C), lambda b: (0, 0)),
            pl.BlockSpec((1, 3 * C), lambda b: (0, 0)),
            pl.BlockSpec((C, NH), lambda b: (0, 0)),
            pl.BlockSpec((C, C), lambda b: (0, 0)),
            pl.BlockSpec((1, C), lambda b: (0, 0)),
        ],
        out_specs=[
            pl.BlockSpec((1, _KP, C), lambda b: (b, 0, 0)),
            pl.BlockSpec((1, 1, _TP), lambda b: (b, 0, 0)),
        ],
        out_shape=[
            jax.ShapeDtypeStruct((B, _KP, C), f32),
            jax.ShapeDtypeStruct((B, 1, _TP), jnp.int32),
        ],
        compiler_params=pltpu.CompilerParams(
            dimension_semantics=("arbitrary",)),
    )(x, W_qkv, b_qkv2, head_onehot, W_proj, b_proj2)


def _sc_unmerge_call(ytab, uidx_flat, B, T):
    C = ytab.shape[1]
    NC = 2
    CH = 72
    TO = 584
    sizes = [CH] * (TO // CH) + ([TO % CH] if TO % CH else [])
    offs = [CH * i for i in range(len(sizes))]
    mesh = plsc.VectorSubcoreMesh(core_axis_name="c", subcore_axis_name="s")

    @functools.partial(
        pl.kernel, mesh=mesh,
        out_type=jax.ShapeDtypeStruct((B, TO, C), jnp.float32),
        scratch_types=[
            pltpu.VMEM((_TP,), jnp.int32),
            pltpu.VMEM((CH, C), jnp.float32),
            pltpu.VMEM((CH, C), jnp.float32),
            pltpu.SemaphoreType.DMA,
            pltpu.SemaphoreType.DMA,
        ],
    )
    def unmerge_k(ytab_hbm, uidx_hbm, out_hbm, idxb_v, rows_a, rows_b,
                  sem_a, sem_b):
        cid = lax.axis_index("c")
        sid = lax.axis_index("s")
        b = sid * NC + cid
        pltpu.sync_copy(
            uidx_hbm.at[pl.ds(pl.multiple_of(b * _TP, 8), _TP)], idxb_v)
        bufs = [rows_a, rows_b]
        sems = [sem_a, sem_b]

        def gather(c):
            return pltpu.async_copy(
                ytab_hbm.at[idxb_v.at[pl.ds(offs[c], sizes[c])]],
                bufs[c % 2].at[pl.ds(0, sizes[c])], sems[c % 2])

        cur = gather(0)
        for c in range(len(sizes)):
            nxt = gather(c + 1) if c + 1 < len(sizes) else None
            cur.wait()
            pltpu.sync_copy(bufs[c % 2].at[pl.ds(0, sizes[c])],
                            out_hbm.at[b, pl.ds(offs[c], sizes[c])])
            cur = nxt

    return unmerge_k(ytab, uidx_flat)


def _run(x, W_qkv, b_qkv, W_proj, b_proj, H, R):
    B, T, C = x.shape
    y, uidx = _fused_call(x, W_qkv, b_qkv, W_proj, b_proj, H, R)
    out = _sc_unmerge_call(y.reshape(B * _KP, C), uidx.reshape(-1), B, T)
    return out[:, :T, :]


def kernel(x, W_qkv, b_qkv, W_proj, b_proj, layer_idx, total_layers):
    return _run(x, W_qkv, b_qkv, W_proj, b_proj, H=12, R=128)
